# baseline (speedup 1.0000x reference)
import functools

import jax
import jax.numpy as jnp
from jax import lax
from jax.experimental import pallas as pl
from jax.experimental.pallas import tpu as pltpu
from jax.experimental.pallas import tpu_sc as plsc

N = 10000
D = 256
H = 128
E = 160000
NC, NS = 2, 16
CH = 128
EPAD = 163840
CPT = EPAD // (NS * CH)
EPT = EPAD // NS
NCHUNK = EPAD // CH
K1CPT = NCHUNK // (NC * NS)
NPAD = 10240
DPT = NPAD // NS

_mesh = plsc.VectorSubcoreMesh(core_axis_name="c", subcore_axis_name="s")


K1EPT = EPAD // (NC * NS)


@functools.partial(
    pl.kernel,
    out_type=jax.ShapeDtypeStruct((NC, NPAD), jnp.float32),
    mesh=_mesh,
    compiler_params=pltpu.CompilerParams(needs_layout_passes=False),
    scratch_types=[
        pltpu.VMEM((K1EPT,), jnp.int32),
        pltpu.VMEM((NPAD,), jnp.float32),
        pltpu.VMEM((NS, DPT), jnp.float32),
        pltpu.VMEM((DPT,), jnp.float32),
        pltpu.VMEM_SHARED((NS, NPAD), jnp.float32),
    ],
)
def _deg_kernel(dst_hbm, zeros_hbm, out_hbm, idx_v, hist_v, red_v, out_v, sh_s):
    c = lax.axis_index("c")
    s = lax.axis_index("s")
    base = c * (NS * K1EPT) + s * K1EPT
    pltpu.sync_copy(dst_hbm.at[pl.ds(base, K1EPT)], idx_v)
    pltpu.sync_copy(zeros_hbm, hist_v)
    ones = jnp.ones((16,), jnp.float32)

    def body(j, carry):
        eb = j * 128
        for k in range(8):
            idx = idx_v[pl.ds(eb + k * 16, 16)]
            plsc.addupdate_scatter(hist_v, (idx,), ones)
        return carry

    lax.fori_loop(0, K1EPT // 128, body, 0)

    pltpu.sync_copy(hist_v, sh_s.at[s])
    plsc.subcore_barrier()
    for t in range(NS):
        pltpu.sync_copy(sh_s.at[t, pl.ds(s * DPT, DPT)], red_v.at[t])

    def rbody(k, carry):
        lb = k * 16
        tot = red_v[0, pl.ds(lb, 16)]
        for t in range(1, NS):
            tot = tot + red_v[t, pl.ds(lb, 16)]
        out_v[pl.ds(lb, 16)] = tot
        return carry

    lax.fori_loop(0, DPT // 16, rbody, 0)
    pltpu.sync_copy(out_v, out_hbm.at[c, pl.ds(s * DPT, DPT)])


def _mm_body(x_ref, wl_ref, wg_ref, xw_ref):
    h1 = jnp.dot(x_ref[...], wl_ref[...], preferred_element_type=jnp.float32)
    xw = jnp.dot(h1, wg_ref[...], preferred_element_type=jnp.float32)
    xw_ref[0] = xw[:, :H]
    xw_ref[1] = xw[:, H:]


_R2 = 1000


def _mm_call(x, W_lin, W_gcn):
    grid = N // _R2
    return pl.pallas_call(
        _mm_body,
        grid=(grid,),
        in_specs=[
            pl.BlockSpec((_R2, D), lambda i: (i, 0)),
            pl.BlockSpec((D, D), lambda i: (0, 0)),
            pl.BlockSpec((D, D), lambda i: (0, 0)),
        ],
        out_specs=pl.BlockSpec((NC, _R2, H), lambda i: (0, i, 0)),
        out_shape=jax.ShapeDtypeStruct((NC, NPAD, H), jnp.float32),
    )(x, W_lin, W_gcn)


def _scale_body(xw_ref, deg_ref, y_ref):
    dinv = jnp.broadcast_to(lax.rsqrt(deg_ref[...])[:, 0:1], (_R2, H))
    y_ref[0] = xw_ref[0] * dinv
    y_ref[1] = xw_ref[1] * dinv


def _scale_call(xw3, degp):
    grid = N // _R2
    return pl.pallas_call(
        _scale_body,
        grid=(grid,),
        in_specs=[
            pl.BlockSpec((NC, _R2, H), lambda i: (0, i, 0)),
            pl.BlockSpec((_R2, 8), lambda i: (i, 0)),
        ],
        out_specs=pl.BlockSpec((NC, _R2, H), lambda i: (0, i, 0)),
        out_shape=jax.ShapeDtypeStruct((NC, NPAD, H), jnp.float32),
    )(xw3, degp)


@functools.partial(
    pl.kernel,
    out_type=jax.ShapeDtypeStruct((NC, NPAD, H), jnp.float32),
    mesh=_mesh,
    scratch_types=[
        pltpu.VMEM((EPT // 2,), jnp.int32),
        pltpu.VMEM((CPT // 2, CH), jnp.int32),
        pltpu.VMEM((CH, H), jnp.float32),
        pltpu.VMEM((CH, H), jnp.float32),
        pltpu.VMEM_SHARED((NPAD, H), jnp.float32),
        pltpu.SemaphoreType.DMA,
        pltpu.SemaphoreType.DMA,
        pltpu.SemaphoreType.DMA,
        pltpu.SemaphoreType.DMA,
    ],
)
def _scat_kernel(y_hbm, src_hbm, dst_hbm, out_hbm,
                 src_v, dst_v, buf0, buf1, acc_s,
                 gsem0, gsem1, ssem0, ssem1):
    c = lax.axis_index("c")
    s = lax.axis_index("s")
    pltpu.sync_copy(y_hbm.at[pl.ds(c * NPAD + s * DPT, DPT)],
                    acc_s.at[pl.ds(s * DPT, DPT)])
    plsc.subcore_barrier()

    bufs = (buf0, buf1)
    gsems = (gsem0, gsem1)
    ssems = (ssem0, ssem1)
    hcpt = CPT // 2
    for hf in range(2):
        pltpu.sync_copy(
            src_hbm.at[pl.ds(c * EPAD + s * EPT + hf * (EPT // 2), EPT // 2)],
            src_v)
        pltpu.sync_copy(dst_hbm.at[pl.ds(s * CPT + hf * hcpt, hcpt)], dst_v)

        def body(i, carry, hf=hf):
            gs = []
            for b in range(2):
                l = i * 4 + b
                gs.append(pltpu.async_copy(
                    y_hbm.at[src_v.at[pl.ds(l * CH, CH)]], bufs[b], gsems[b]))
            ss = []
            for b in range(2):
                l = i * 4 + b
                gs[b].wait()
                ss.append(pltpu.async_copy(
                    bufs[b], acc_s.at[dst_v.at[l]], ssems[b], add=True))
            gs2 = []
            for b in range(2):
                l = i * 4 + 2 + b
                ss[b].wait()
                gs2.append(pltpu.async_copy(
                    y_hbm.at[src_v.at[pl.ds(l * CH, CH)]], bufs[b], gsems[b]))
            ss2 = []
            for b in range(2):
                l = i * 4 + 2 + b
                gs2[b].wait()
                ss2.append(pltpu.async_copy(
                    bufs[b], acc_s.at[dst_v.at[l]], ssems[b], add=True))
            for b in range(2):
                ss2[b].wait()
            return carry

        lax.fori_loop(0, hcpt // 4, body, 0)
    plsc.subcore_barrier()
    pltpu.sync_copy(acc_s.at[pl.ds(s * DPT, DPT)],
                    out_hbm.at[c, pl.ds(s * DPT, DPT)])


_R4 = 1000


def _fin_body(acc_ref, deg_ref, b_ref, g_ref, be_ref, o_ref):
    dinv = jnp.broadcast_to(lax.rsqrt(deg_ref[...])[:, 0:1], (_R4, H))
    h = jnp.concatenate([acc_ref[0] * dinv, acc_ref[1] * dinv], axis=-1)
    h = h + b_ref[...]
    mu = jnp.mean(h, axis=-1, keepdims=True)
    xc = h - mu
    var = jnp.mean(xc * xc, axis=-1, keepdims=True)
    hn = xc * lax.rsqrt(var + 1e-5) * g_ref[...] + be_ref[...]
    o_ref[...] = jnp.maximum(hn, 0.0)


def _fin_call(acc, degp, b2, g2, be2):
    grid = N // _R4
    return pl.pallas_call(
        _fin_body,
        grid=(grid,),
        in_specs=[
            pl.BlockSpec((NC, _R4, H), lambda i: (0, i, 0)),
            pl.BlockSpec((_R4, 8), lambda i: (i, 0)),
            pl.BlockSpec((1, D), lambda i: (0, 0)),
            pl.BlockSpec((1, D), lambda i: (0, 0)),
            pl.BlockSpec((1, D), lambda i: (0, 0)),
        ],
        out_specs=pl.BlockSpec((_R4, D), lambda i: (i, 0)),
        out_shape=jax.ShapeDtypeStruct((N, D), jnp.float32),
    )(acc, degp, b2, g2, be2)


def kernel(x, edge_index, W_lin, W_gcn, b_gcn, gamma, beta):
    src = edge_index[0].astype(jnp.int32)
    dst = edge_index[1].astype(jnp.int32)
    npad = EPAD - E
    src_p = jnp.concatenate([src, jnp.full((npad,), N, jnp.int32)])
    dst_p = jnp.concatenate([dst, jnp.full((npad,), N, jnp.int32)])
    src2 = jnp.concatenate([src_p, src_p + NPAD])
    dst2d = dst_p.reshape(NCHUNK, CH)

    zeros = jnp.zeros((NPAD,), jnp.float32)
    degp = _deg_kernel(dst_p, zeros)
    deg8 = jnp.broadcast_to(
        (degp[0, :N] + degp[1, :N] + 1.0)[:, None], (N, 8))

    xw3 = _mm_call(x, W_lin, W_gcn)
    y3 = _scale_call(xw3, deg8)
    y_flat = y3.reshape(2 * NPAD, H)

    acc = _scat_kernel(y_flat, src2, dst2d)

    return _fin_call(acc, deg8, b_gcn[None, :], gamma[None, :], beta[None, :])


# speedup vs baseline: 1.0917x; 1.0917x over previous
import functools

import jax
import jax.numpy as jnp
from jax import lax
from jax.experimental import pallas as pl
from jax.experimental.pallas import tpu as pltpu
from jax.experimental.pallas import tpu_sc as plsc

N = 10000
D = 256
H = 128
E = 160000
NC, NS = 2, 16
CH = 64
EPAD = 163840
CPT = EPAD // (NS * CH)
EPT = EPAD // NS
NCHUNK = EPAD // CH
K1CPT = NCHUNK // (NC * NS)
NPAD = 10240
DPT = NPAD // NS

_mesh = plsc.VectorSubcoreMesh(core_axis_name="c", subcore_axis_name="s")


K1EPT = EPAD // (NC * NS)


@functools.partial(
    pl.kernel,
    out_type=jax.ShapeDtypeStruct((NC, NPAD), jnp.float32),
    mesh=_mesh,
    compiler_params=pltpu.CompilerParams(needs_layout_passes=False),
    scratch_types=[
        pltpu.VMEM((K1EPT,), jnp.int32),
        pltpu.VMEM((NPAD,), jnp.float32),
        pltpu.VMEM((NS, DPT), jnp.float32),
        pltpu.VMEM((DPT,), jnp.float32),
        pltpu.VMEM_SHARED((NS, NPAD), jnp.float32),
    ],
)
def _deg_kernel(dst_hbm, zeros_hbm, out_hbm, idx_v, hist_v, red_v, out_v, sh_s):
    c = lax.axis_index("c")
    s = lax.axis_index("s")
    base = c * (NS * K1EPT) + s * K1EPT
    pltpu.sync_copy(dst_hbm.at[pl.ds(base, K1EPT)], idx_v)
    pltpu.sync_copy(zeros_hbm, hist_v)
    ones = jnp.ones((16,), jnp.float32)

    def body(j, carry):
        eb = j * 128
        for k in range(8):
            idx = idx_v[pl.ds(eb + k * 16, 16)]
            plsc.addupdate_scatter(hist_v, (idx,), ones)
        return carry

    lax.fori_loop(0, K1EPT // 128, body, 0)

    pltpu.sync_copy(hist_v, sh_s.at[s])
    plsc.subcore_barrier()
    for t in range(NS):
        pltpu.sync_copy(sh_s.at[t, pl.ds(s * DPT, DPT)], red_v.at[t])

    def rbody(k, carry):
        lb = k * 16
        tot = red_v[0, pl.ds(lb, 16)]
        for t in range(1, NS):
            tot = tot + red_v[t, pl.ds(lb, 16)]
        out_v[pl.ds(lb, 16)] = tot
        return carry

    lax.fori_loop(0, DPT // 16, rbody, 0)
    pltpu.sync_copy(out_v, out_hbm.at[c, pl.ds(s * DPT, DPT)])


def _mm_body(x_ref, wl_ref, wg_ref, xw_ref):
    h1 = jnp.dot(x_ref[...], wl_ref[...], preferred_element_type=jnp.float32)
    xw = jnp.dot(h1, wg_ref[...], preferred_element_type=jnp.float32)
    xw_ref[0] = xw[:, :H]
    xw_ref[1] = xw[:, H:]


_R2 = 1000


def _mm_call(x, W_lin, W_gcn):
    grid = N // _R2
    return pl.pallas_call(
        _mm_body,
        grid=(grid,),
        in_specs=[
            pl.BlockSpec((_R2, D), lambda i: (i, 0)),
            pl.BlockSpec((D, D), lambda i: (0, 0)),
            pl.BlockSpec((D, D), lambda i: (0, 0)),
        ],
        out_specs=pl.BlockSpec((NC, _R2, H), lambda i: (0, i, 0)),
        out_shape=jax.ShapeDtypeStruct((NC, NPAD, H), jnp.float32),
    )(x, W_lin, W_gcn)


def _scale_body(xw_ref, deg_ref, y_ref):
    dinv = jnp.broadcast_to(lax.rsqrt(deg_ref[...])[:, 0:1], (_R2, H))
    y_ref[0] = xw_ref[0] * dinv
    y_ref[1] = xw_ref[1] * dinv


def _scale_call(xw3, degp):
    grid = N // _R2
    return pl.pallas_call(
        _scale_body,
        grid=(grid,),
        in_specs=[
            pl.BlockSpec((NC, _R2, H), lambda i: (0, i, 0)),
            pl.BlockSpec((_R2, 8), lambda i: (i, 0)),
        ],
        out_specs=pl.BlockSpec((NC, _R2, H), lambda i: (0, i, 0)),
        out_shape=jax.ShapeDtypeStruct((NC, NPAD, H), jnp.float32),
    )(xw3, degp)


@functools.partial(
    pl.kernel,
    out_type=jax.ShapeDtypeStruct((NC, NPAD, H), jnp.float32),
    mesh=_mesh,
    scratch_types=[
        pltpu.VMEM((EPT // 2,), jnp.int32),
        pltpu.VMEM((CPT // 2, CH), jnp.int32),
        pltpu.VMEM((CH, H), jnp.float32),
        pltpu.VMEM((CH, H), jnp.float32),
        pltpu.VMEM((CH, H), jnp.float32),
        pltpu.VMEM((CH, H), jnp.float32),
        pltpu.VMEM_SHARED((NPAD, H), jnp.float32),
        pltpu.SemaphoreType.DMA,
        pltpu.SemaphoreType.DMA,
        pltpu.SemaphoreType.DMA,
        pltpu.SemaphoreType.DMA,
        pltpu.SemaphoreType.DMA,
        pltpu.SemaphoreType.DMA,
        pltpu.SemaphoreType.DMA,
        pltpu.SemaphoreType.DMA,
    ],
)
def _scat_kernel(y_hbm, src_hbm, dst_hbm, out_hbm,
                 src_v, dst_v, buf0, buf1, buf2, buf3, acc_s,
                 gsem0, gsem1, gsem2, gsem3, ssem0, ssem1, ssem2, ssem3):
    c = lax.axis_index("c")
    s = lax.axis_index("s")
    pltpu.sync_copy(y_hbm.at[pl.ds(c * NPAD + s * DPT, DPT)],
                    acc_s.at[pl.ds(s * DPT, DPT)])
    plsc.subcore_barrier()

    bufs = (buf0, buf1, buf2, buf3)
    gsems = (gsem0, gsem1, gsem2, gsem3)
    ssems = (ssem0, ssem1, ssem2, ssem3)
    hcpt = CPT // 2
    for hf in range(2):
        pltpu.sync_copy(
            src_hbm.at[pl.ds(c * EPAD + s * EPT + hf * (EPT // 2), EPT // 2)],
            src_v)
        pltpu.sync_copy(dst_hbm.at[pl.ds(s * CPT + hf * hcpt, hcpt)], dst_v)

        def body(i, carry, hf=hf):
            prev_ss = None
            last_ss = None
            for tr in range(4):
                gs = []
                for b in range(4):
                    l = i * 16 + tr * 4 + b
                    if prev_ss is not None:
                        prev_ss[b].wait()
                    gs.append(pltpu.async_copy(
                        y_hbm.at[src_v.at[pl.ds(l * CH, CH)]],
                        bufs[b], gsems[b]))
                ss = []
                for b in range(4):
                    l = i * 16 + tr * 4 + b
                    gs[b].wait()
                    ss.append(pltpu.async_copy(
                        bufs[b], acc_s.at[dst_v.at[l]], ssems[b], add=True))
                prev_ss = ss
                last_ss = ss
            for b in range(4):
                last_ss[b].wait()
            return carry

        lax.fori_loop(0, hcpt // 16, body, 0)
    plsc.subcore_barrier()
    pltpu.sync_copy(acc_s.at[pl.ds(s * DPT, DPT)],
                    out_hbm.at[c, pl.ds(s * DPT, DPT)])


_R4 = 1000


def _fin_body(acc_ref, deg_ref, b_ref, g_ref, be_ref, o_ref):
    dinv = jnp.broadcast_to(lax.rsqrt(deg_ref[...])[:, 0:1], (_R4, H))
    h = jnp.concatenate([acc_ref[0] * dinv, acc_ref[1] * dinv], axis=-1)
    h = h + b_ref[...]
    mu = jnp.mean(h, axis=-1, keepdims=True)
    xc = h - mu
    var = jnp.mean(xc * xc, axis=-1, keepdims=True)
    hn = xc * lax.rsqrt(var + 1e-5) * g_ref[...] + be_ref[...]
    o_ref[...] = jnp.maximum(hn, 0.0)


def _fin_call(acc, degp, b2, g2, be2):
    grid = N // _R4
    return pl.pallas_call(
        _fin_body,
        grid=(grid,),
        in_specs=[
            pl.BlockSpec((NC, _R4, H), lambda i: (0, i, 0)),
            pl.BlockSpec((_R4, 8), lambda i: (i, 0)),
            pl.BlockSpec((1, D), lambda i: (0, 0)),
            pl.BlockSpec((1, D), lambda i: (0, 0)),
            pl.BlockSpec((1, D), lambda i: (0, 0)),
        ],
        out_specs=pl.BlockSpec((_R4, D), lambda i: (i, 0)),
        out_shape=jax.ShapeDtypeStruct((N, D), jnp.float32),
    )(acc, degp, b2, g2, be2)


def kernel(x, edge_index, W_lin, W_gcn, b_gcn, gamma, beta):
    src = edge_index[0].astype(jnp.int32)
    dst = edge_index[1].astype(jnp.int32)
    npad = EPAD - E
    src_p = jnp.concatenate([src, jnp.full((npad,), N, jnp.int32)])
    dst_p = jnp.concatenate([dst, jnp.full((npad,), N, jnp.int32)])
    src2 = jnp.concatenate([src_p, src_p + NPAD])
    dst2d = dst_p.reshape(NCHUNK, CH)

    zeros = jnp.zeros((NPAD,), jnp.float32)
    degp = _deg_kernel(dst_p, zeros)
    deg8 = jnp.broadcast_to(
        (degp[0, :N] + degp[1, :N] + 1.0)[:, None], (N, 8))

    xw3 = _mm_call(x, W_lin, W_gcn)
    y3 = _scale_call(xw3, deg8)
    y_flat = y3.reshape(2 * NPAD, H)

    acc = _scat_kernel(y_flat, src2, dst2d)

    return _fin_call(acc, deg8, b_gcn[None, :], gamma[None, :], beta[None, :])


# speedup vs baseline: 1.0946x; 1.0027x over previous
import functools

import jax
import jax.numpy as jnp
from jax import lax
from jax.experimental import pallas as pl
from jax.experimental.pallas import tpu as pltpu
from jax.experimental.pallas import tpu_sc as plsc

N = 10000
D = 256
H = 128
E = 160000
NC, NS = 2, 16
CH = 64
EPAD = 163840
CPT = EPAD // (NS * CH)
EPT = EPAD // NS
NCHUNK = EPAD // CH
K1CPT = NCHUNK // (NC * NS)
NPAD = 10240
DPT = NPAD // NS

_mesh = plsc.VectorSubcoreMesh(core_axis_name="c", subcore_axis_name="s")


K1EPT = EPAD // (NC * NS)


@functools.partial(
    pl.kernel,
    out_type=jax.ShapeDtypeStruct((NC, NPAD), jnp.float32),
    mesh=_mesh,
    compiler_params=pltpu.CompilerParams(needs_layout_passes=False),
    scratch_types=[
        pltpu.VMEM((K1EPT,), jnp.int32),
        pltpu.VMEM((NPAD,), jnp.float32),
        pltpu.VMEM((NS, DPT), jnp.float32),
        pltpu.VMEM((DPT,), jnp.float32),
        pltpu.VMEM_SHARED((NS, NPAD), jnp.float32),
    ],
)
def _deg_kernel(dst_hbm, zeros_hbm, out_hbm, idx_v, hist_v, red_v, out_v, sh_s):
    c = lax.axis_index("c")
    s = lax.axis_index("s")
    base = c * (NS * K1EPT) + s * K1EPT
    pltpu.sync_copy(dst_hbm.at[pl.ds(base, K1EPT)], idx_v)
    pltpu.sync_copy(zeros_hbm, hist_v)
    ones = jnp.ones((16,), jnp.float32)

    def body(j, carry):
        eb = j * 128
        for k in range(8):
            idx = idx_v[pl.ds(eb + k * 16, 16)]
            plsc.addupdate_scatter(hist_v, (idx,), ones)
        return carry

    lax.fori_loop(0, K1EPT // 128, body, 0)

    pltpu.sync_copy(hist_v, sh_s.at[s])
    plsc.subcore_barrier()
    for t in range(NS):
        pltpu.sync_copy(sh_s.at[t, pl.ds(s * DPT, DPT)], red_v.at[t])

    def rbody(k, carry):
        lb = k * 16
        tot = red_v[0, pl.ds(lb, 16)]
        for t in range(1, NS):
            tot = tot + red_v[t, pl.ds(lb, 16)]
        out_v[pl.ds(lb, 16)] = tot
        return carry

    lax.fori_loop(0, DPT // 16, rbody, 0)
    pltpu.sync_copy(out_v, out_hbm.at[c, pl.ds(s * DPT, DPT)])


def _mm_body(x_ref, wl_ref, wg_ref, xw_ref):
    wc = jnp.dot(wl_ref[...], wg_ref[...], preferred_element_type=jnp.float32)
    xw = jnp.dot(x_ref[...], wc, preferred_element_type=jnp.float32)
    xw_ref[0] = xw[:, :H]
    xw_ref[1] = xw[:, H:]


_R2 = 1000


def _mm_call(x, W_lin, W_gcn):
    grid = N // _R2
    return pl.pallas_call(
        _mm_body,
        grid=(grid,),
        in_specs=[
            pl.BlockSpec((_R2, D), lambda i: (i, 0)),
            pl.BlockSpec((D, D), lambda i: (0, 0)),
            pl.BlockSpec((D, D), lambda i: (0, 0)),
        ],
        out_specs=pl.BlockSpec((NC, _R2, H), lambda i: (0, i, 0)),
        out_shape=jax.ShapeDtypeStruct((NC, NPAD, H), jnp.float32),
    )(x, W_lin, W_gcn)


def _scale_body(xw_ref, deg_ref, y_ref):
    dinv = jnp.broadcast_to(lax.rsqrt(deg_ref[...])[:, 0:1], (_R2, H))
    y_ref[0] = xw_ref[0] * dinv
    y_ref[1] = xw_ref[1] * dinv


def _scale_call(xw3, degp):
    grid = N // _R2
    return pl.pallas_call(
        _scale_body,
        grid=(grid,),
        in_specs=[
            pl.BlockSpec((NC, _R2, H), lambda i: (0, i, 0)),
            pl.BlockSpec((_R2, 8), lambda i: (i, 0)),
        ],
        out_specs=pl.BlockSpec((NC, _R2, H), lambda i: (0, i, 0)),
        out_shape=jax.ShapeDtypeStruct((NC, NPAD, H), jnp.float32),
    )(xw3, degp)


@functools.partial(
    pl.kernel,
    out_type=jax.ShapeDtypeStruct((NC, NPAD, H), jnp.float32),
    mesh=_mesh,
    scratch_types=[
        pltpu.VMEM((EPT // 2,), jnp.int32),
        pltpu.VMEM((CPT // 2, CH), jnp.int32),
        pltpu.VMEM((CH, H), jnp.float32),
        pltpu.VMEM((CH, H), jnp.float32),
        pltpu.VMEM((CH, H), jnp.float32),
        pltpu.VMEM((CH, H), jnp.float32),
        pltpu.VMEM_SHARED((NPAD, H), jnp.float32),
        pltpu.SemaphoreType.DMA,
        pltpu.SemaphoreType.DMA,
        pltpu.SemaphoreType.DMA,
        pltpu.SemaphoreType.DMA,
        pltpu.SemaphoreType.DMA,
        pltpu.SemaphoreType.DMA,
        pltpu.SemaphoreType.DMA,
        pltpu.SemaphoreType.DMA,
    ],
)
def _scat_kernel(y_hbm, src_hbm, dst_hbm, out_hbm,
                 src_v, dst_v, buf0, buf1, buf2, buf3, acc_s,
                 gsem0, gsem1, gsem2, gsem3, ssem0, ssem1, ssem2, ssem3):
    c = lax.axis_index("c")
    s = lax.axis_index("s")
    ini = pltpu.async_copy(y_hbm.at[pl.ds(c * NPAD + s * DPT, DPT)],
                           acc_s.at[pl.ds(s * DPT, DPT)], gsem0)

    bufs = (buf0, buf1, buf2, buf3)
    gsems = (gsem0, gsem1, gsem2, gsem3)
    ssems = (ssem0, ssem1, ssem2, ssem3)
    hcpt = CPT // 2
    for hf in range(2):
        pltpu.sync_copy(
            src_hbm.at[pl.ds(c * EPAD + s * EPT + hf * (EPT // 2), EPT // 2)],
            src_v)
        pltpu.sync_copy(dst_hbm.at[pl.ds(s * CPT + hf * hcpt, hcpt)], dst_v)
        if hf == 0:
            ini.wait()
            plsc.subcore_barrier()

        def body(i, carry, hf=hf):
            prev_ss = None
            last_ss = None
            for tr in range(4):
                gs = []
                for b in range(4):
                    l = i * 16 + tr * 4 + b
                    if prev_ss is not None:
                        prev_ss[b].wait()
                    gs.append(pltpu.async_copy(
                        y_hbm.at[src_v.at[pl.ds(l * CH, CH)]],
                        bufs[b], gsems[b]))
                ss = []
                for b in range(4):
                    l = i * 16 + tr * 4 + b
                    gs[b].wait()
                    ss.append(pltpu.async_copy(
                        bufs[b], acc_s.at[dst_v.at[l]], ssems[b], add=True))
                prev_ss = ss
                last_ss = ss
            for b in range(4):
                last_ss[b].wait()
            return carry

        lax.fori_loop(0, hcpt // 16, body, 0)
    plsc.subcore_barrier()
    pltpu.sync_copy(acc_s.at[pl.ds(s * DPT, DPT)],
                    out_hbm.at[c, pl.ds(s * DPT, DPT)])


_R4 = 1000


def _fin_body(acc_ref, deg_ref, b_ref, g_ref, be_ref, o_ref):
    dinv = jnp.broadcast_to(lax.rsqrt(deg_ref[...])[:, 0:1], (_R4, H))
    h = jnp.concatenate([acc_ref[0] * dinv, acc_ref[1] * dinv], axis=-1)
    h = h + b_ref[...]
    mu = jnp.mean(h, axis=-1, keepdims=True)
    xc = h - mu
    var = jnp.mean(xc * xc, axis=-1, keepdims=True)
    hn = xc * lax.rsqrt(var + 1e-5) * g_ref[...] + be_ref[...]
    o_ref[...] = jnp.maximum(hn, 0.0)


def _fin_call(acc, degp, b2, g2, be2):
    grid = N // _R4
    return pl.pallas_call(
        _fin_body,
        grid=(grid,),
        in_specs=[
            pl.BlockSpec((NC, _R4, H), lambda i: (0, i, 0)),
            pl.BlockSpec((_R4, 8), lambda i: (i, 0)),
            pl.BlockSpec((1, D), lambda i: (0, 0)),
            pl.BlockSpec((1, D), lambda i: (0, 0)),
            pl.BlockSpec((1, D), lambda i: (0, 0)),
        ],
        out_specs=pl.BlockSpec((_R4, D), lambda i: (i, 0)),
        out_shape=jax.ShapeDtypeStruct((N, D), jnp.float32),
    )(acc, degp, b2, g2, be2)


def kernel(x, edge_index, W_lin, W_gcn, b_gcn, gamma, beta):
    src = edge_index[0].astype(jnp.int32)
    dst = edge_index[1].astype(jnp.int32)
    npad = EPAD - E
    src_p = jnp.concatenate([src, jnp.full((npad,), N, jnp.int32)])
    dst_p = jnp.concatenate([dst, jnp.full((npad,), N, jnp.int32)])
    src2 = jnp.concatenate([src_p, src_p + NPAD])
    dst2d = dst_p.reshape(NCHUNK, CH)

    zeros = jnp.zeros((NPAD,), jnp.float32)
    degp = _deg_kernel(dst_p, zeros)
    deg8 = jnp.broadcast_to(
        (degp[0, :N] + degp[1, :N] + 1.0)[:, None], (N, 8))

    xw3 = _mm_call(x, W_lin, W_gcn)
    y3 = _scale_call(xw3, deg8)
    y_flat = y3.reshape(2 * NPAD, H)

    acc = _scat_kernel(y_flat, src2, dst2d)

    return _fin_call(acc, deg8, b_gcn[None, :], gamma[None, :], beta[None, :])


# speedup vs baseline: 1.0952x; 1.0005x over previous
import functools

import jax
import jax.numpy as jnp
from jax import lax
from jax.experimental import pallas as pl
from jax.experimental.pallas import tpu as pltpu
from jax.experimental.pallas import tpu_sc as plsc

N = 10000
D = 256
H = 128
E = 160000
NC, NS = 2, 16
CH = 64
EPAD = 163840
CPT = EPAD // (NS * CH)
EPT = EPAD // NS
NCHUNK = EPAD // CH
K1CPT = NCHUNK // (NC * NS)
NPAD = 10240
DPT = NPAD // NS

_mesh = plsc.VectorSubcoreMesh(core_axis_name="c", subcore_axis_name="s")


K1EPT = EPAD // (NC * NS)


@functools.partial(
    pl.kernel,
    out_type=jax.ShapeDtypeStruct((NC, NPAD), jnp.float32),
    mesh=_mesh,
    compiler_params=pltpu.CompilerParams(needs_layout_passes=False),
    scratch_types=[
        pltpu.VMEM((K1EPT,), jnp.int32),
        pltpu.VMEM((NPAD,), jnp.float32),
        pltpu.VMEM((NS, DPT), jnp.float32),
        pltpu.VMEM((DPT,), jnp.float32),
        pltpu.VMEM_SHARED((NS, NPAD), jnp.float32),
    ],
)
def _deg_kernel(dst_hbm, zeros_hbm, out_hbm, idx_v, hist_v, red_v, out_v, sh_s):
    c = lax.axis_index("c")
    s = lax.axis_index("s")
    base = c * (NS * K1EPT) + s * K1EPT
    pltpu.sync_copy(dst_hbm.at[pl.ds(base, K1EPT)], idx_v)
    pltpu.sync_copy(zeros_hbm, hist_v)
    ones = jnp.ones((16,), jnp.float32)

    def body(j, carry):
        eb = j * 128
        for k in range(8):
            idx = idx_v[pl.ds(eb + k * 16, 16)]
            plsc.addupdate_scatter(hist_v, (idx,), ones)
        return carry

    lax.fori_loop(0, K1EPT // 128, body, 0)

    pltpu.sync_copy(hist_v, sh_s.at[s])
    plsc.subcore_barrier()
    for t in range(NS):
        pltpu.sync_copy(sh_s.at[t, pl.ds(s * DPT, DPT)], red_v.at[t])

    def rbody(k, carry):
        lb = k * 16
        tot = red_v[0, pl.ds(lb, 16)]
        for t in range(1, NS):
            tot = tot + red_v[t, pl.ds(lb, 16)]
        out_v[pl.ds(lb, 16)] = tot
        return carry

    lax.fori_loop(0, DPT // 16, rbody, 0)
    pltpu.sync_copy(out_v, out_hbm.at[c, pl.ds(s * DPT, DPT)])


def _mm_body(x_ref, wl_ref, wg_ref, xw_ref):
    h1 = jnp.dot(x_ref[...], wl_ref[...], preferred_element_type=jnp.float32)
    xw = jnp.dot(h1, wg_ref[...], preferred_element_type=jnp.float32)
    xw_ref[0] = xw[:, :H]
    xw_ref[1] = xw[:, H:]


_R2 = 1000


def _mm_call(x, W_lin, W_gcn):
    grid = N // _R2
    return pl.pallas_call(
        _mm_body,
        grid=(grid,),
        in_specs=[
            pl.BlockSpec((_R2, D), lambda i: (i, 0)),
            pl.BlockSpec((D, D), lambda i: (0, 0)),
            pl.BlockSpec((D, D), lambda i: (0, 0)),
        ],
        out_specs=pl.BlockSpec((NC, _R2, H), lambda i: (0, i, 0)),
        out_shape=jax.ShapeDtypeStruct((NC, NPAD, H), jnp.float32),
    )(x, W_lin, W_gcn)


def _scale_body(xw_ref, deg_ref, y_ref):
    dinv = jnp.broadcast_to(lax.rsqrt(deg_ref[...])[:, 0:1], (_R2, H))
    y_ref[0] = xw_ref[0] * dinv
    y_ref[1] = xw_ref[1] * dinv


def _scale_call(xw3, degp):
    grid = N // _R2
    return pl.pallas_call(
        _scale_body,
        grid=(grid,),
        in_specs=[
            pl.BlockSpec((NC, _R2, H), lambda i: (0, i, 0)),
            pl.BlockSpec((_R2, 8), lambda i: (i, 0)),
        ],
        out_specs=pl.BlockSpec((NC, _R2, H), lambda i: (0, i, 0)),
        out_shape=jax.ShapeDtypeStruct((NC, NPAD, H), jnp.float32),
    )(xw3, degp)


@functools.partial(
    pl.kernel,
    out_type=jax.ShapeDtypeStruct((NC, NPAD, H), jnp.float32),
    mesh=_mesh,
    scratch_types=[
        pltpu.VMEM((EPT // 2,), jnp.int32),
        pltpu.VMEM((CPT // 2, CH), jnp.int32),
        pltpu.VMEM((CH, H), jnp.float32),
        pltpu.VMEM((CH, H), jnp.float32),
        pltpu.VMEM((CH, H), jnp.float32),
        pltpu.VMEM((CH, H), jnp.float32),
        pltpu.VMEM_SHARED((NPAD, H), jnp.float32),
        pltpu.SemaphoreType.DMA,
        pltpu.SemaphoreType.DMA,
        pltpu.SemaphoreType.DMA,
        pltpu.SemaphoreType.DMA,
        pltpu.SemaphoreType.DMA,
        pltpu.SemaphoreType.DMA,
        pltpu.SemaphoreType.DMA,
        pltpu.SemaphoreType.DMA,
    ],
)
def _scat_kernel(y_hbm, src_hbm, dst_hbm, out_hbm,
                 src_v, dst_v, buf0, buf1, buf2, buf3, acc_s,
                 gsem0, gsem1, gsem2, gsem3, ssem0, ssem1, ssem2, ssem3):
    c = lax.axis_index("c")
    s = lax.axis_index("s")
    ini = pltpu.async_copy(y_hbm.at[pl.ds(c * NPAD + s * DPT, DPT)],
                           acc_s.at[pl.ds(s * DPT, DPT)], gsem0)

    bufs = (buf0, buf1, buf2, buf3)
    gsems = (gsem0, gsem1, gsem2, gsem3)
    ssems = (ssem0, ssem1, ssem2, ssem3)
    hcpt = CPT // 2
    for hf in range(2):
        pltpu.sync_copy(
            src_hbm.at[pl.ds(c * EPAD + s * EPT + hf * (EPT // 2), EPT // 2)],
            src_v)
        pltpu.sync_copy(dst_hbm.at[pl.ds(s * CPT + hf * hcpt, hcpt)], dst_v)
        if hf == 0:
            ini.wait()
            plsc.subcore_barrier()

        def body(i, carry, hf=hf):
            prev_ss = None
            last_ss = None
            for tr in range(4):
                gs = []
                for b in range(4):
                    l = i * 16 + tr * 4 + b
                    if prev_ss is not None:
                        prev_ss[b].wait()
                    gs.append(pltpu.async_copy(
                        y_hbm.at[src_v.at[pl.ds(l * CH, CH)]],
                        bufs[b], gsems[b]))
                ss = []
                for b in range(4):
                    l = i * 16 + tr * 4 + b
                    gs[b].wait()
                    ss.append(pltpu.async_copy(
                        bufs[b], acc_s.at[dst_v.at[l]], ssems[b], add=True))
                prev_ss = ss
                last_ss = ss
            for b in range(4):
                last_ss[b].wait()
            return carry

        lax.fori_loop(0, hcpt // 16, body, 0)
    plsc.subcore_barrier()
    pltpu.sync_copy(acc_s.at[pl.ds(s * DPT, DPT)],
                    out_hbm.at[c, pl.ds(s * DPT, DPT)])


_R4 = 1000


def _fin_body(acc_ref, deg_ref, b_ref, g_ref, be_ref, o_ref):
    dinv = jnp.broadcast_to(lax.rsqrt(deg_ref[...])[:, 0:1], (_R4, H))
    h = jnp.concatenate([acc_ref[0] * dinv, acc_ref[1] * dinv], axis=-1)
    h = h + b_ref[...]
    mu = jnp.mean(h, axis=-1, keepdims=True)
    xc = h - mu
    var = jnp.mean(xc * xc, axis=-1, keepdims=True)
    hn = xc * lax.rsqrt(var + 1e-5) * g_ref[...] + be_ref[...]
    o_ref[...] = jnp.maximum(hn, 0.0)


def _fin_call(acc, degp, b2, g2, be2):
    grid = N // _R4
    return pl.pallas_call(
        _fin_body,
        grid=(grid,),
        in_specs=[
            pl.BlockSpec((NC, _R4, H), lambda i: (0, i, 0)),
            pl.BlockSpec((_R4, 8), lambda i: (i, 0)),
            pl.BlockSpec((1, D), lambda i: (0, 0)),
            pl.BlockSpec((1, D), lambda i: (0, 0)),
            pl.BlockSpec((1, D), lambda i: (0, 0)),
        ],
        out_specs=pl.BlockSpec((_R4, D), lambda i: (i, 0)),
        out_shape=jax.ShapeDtypeStruct((N, D), jnp.float32),
    )(acc, degp, b2, g2, be2)


def kernel(x, edge_index, W_lin, W_gcn, b_gcn, gamma, beta):
    src = edge_index[0].astype(jnp.int32)
    dst = edge_index[1].astype(jnp.int32)
    npad = EPAD - E
    src_p = jnp.concatenate([src, jnp.full((npad,), N, jnp.int32)])
    dst_p = jnp.concatenate([dst, jnp.full((npad,), N, jnp.int32)])
    src2 = jnp.concatenate([src_p, src_p + NPAD])
    dst2d = dst_p.reshape(NCHUNK, CH)

    zeros = jnp.zeros((NPAD,), jnp.float32)
    degp = _deg_kernel(dst_p, zeros)
    deg8 = jnp.broadcast_to(
        (degp[0, :N] + degp[1, :N] + 1.0)[:, None], (N, 8))

    xw3 = _mm_call(x, W_lin, W_gcn)
    y3 = _scale_call(xw3, deg8)
    y_flat = y3.reshape(2 * NPAD, H)

    acc = _scat_kernel(y_flat, src2, dst2d)

    return _fin_call(acc, deg8, b_gcn[None, :], gamma[None, :], beta[None, :])
